# baseline (speedup 1.0000x reference)
import functools

import numpy as np

import jax
import jax.numpy as jnp
from jax import lax
from jax.experimental import pallas as pl
from jax.experimental.pallas import tpu as pltpu
from jax.experimental.pallas import tpu_sc as plsc

N = 10000
E = 160000
T = 8
KT = 3
TA = T - (KT - 1)
C1 = 64
FEAT = TA * C1
FH = FEAT // 2
H0 = 128
OUT = 3

NCORE = 2
NSUB = 16
EPT = E // NSUB
BSTG = 512
NBLK = 20
OVL = NBLK * BSTG - EPT
CHUNK = 8
NCHB = BSTG // CHUNK
GRING = 6
FRING = 3

_QQ = np.arange(64)
PERM64 = 32 * (_QQ // 32) + (_QQ % 32) // 2 + 16 * (_QQ % 2)
ZSLAB = 632
ZLAST = N - (NSUB - 1) * ZSLAB


def _tgc1_body(xt_ref, wct_ref, bc_ref, wa_ref, ba_ref, out_ref):
    for t in range(TA):
        win = xt_ref[:, t:t + KT]
        y = jnp.dot(win, wct_ref[...], preferred_element_type=jnp.float32)
        y = y + bc_ref[...]
        p = y[:, :C1]
        q = y[:, C1:]
        xal = xt_ref[:, t + KT - 1:t + KT] * wa_ref[...] + ba_ref[...]
        th, tt = divmod(t, TA // 2)
        out_ref[th * N:(th + 1) * N, tt * C1:(tt + 1) * C1] = (
            (p + xal) * jax.nn.sigmoid(q)).astype(jnp.bfloat16)


def _sc_agg_body(h_hbm, esrc_hbm, edst_hbm, gso_hbm, out_hbm,
                 src_v, dst_v, gso_v, gbuf, fbuf, acc_sh, sem_g, sem_s):
    c = lax.axis_index("c")
    s = lax.axis_index("s")
    ebase = s * EPT
    goff = c * N

    zf = jnp.zeros((16,), jnp.float32)

    @pl.loop(0, FRING * CHUNK)
    def zrow(r):
        for f in range(FH // 16):
            fbuf[r, pl.ds(f * 16, 16)] = zf

    ZC = FRING * CHUNK

    @pl.when(s < NSUB - 1)
    def _zfull():
        for r0 in range(0, ZSLAB, ZC):
            rl = min(ZC, ZSLAB - r0)
            pltpu.sync_copy(fbuf.at[pl.ds(0, rl)],
                            acc_sh.at[pl.ds(s * ZSLAB + r0, rl)])

    @pl.when(s == NSUB - 1)
    def _zlast():
        for r0 in range(0, ZLAST, ZC):
            rl = min(ZC, ZLAST - r0)
            pltpu.sync_copy(fbuf.at[pl.ds(0, rl)],
                            acc_sh.at[pl.ds(s * ZSLAB + r0, rl)])

    plsc.subcore_barrier()

    @pl.loop(0, NBLK)
    def blk_body(blk):
        bb = ebase + jnp.minimum(blk * BSTG, EPT - BSTG)
        pltpu.sync_copy(esrc_hbm.at[pl.ds(bb, BSTG)], src_v.at[pl.ds(0, BSTG)])
        pltpu.sync_copy(edst_hbm.at[pl.ds(bb, BSTG)], dst_v.at[pl.ds(0, BSTG)])
        pltpu.sync_copy(gso_hbm.at[pl.ds(bb, BSTG)], gso_v.at[pl.ds(0, BSTG)])

        @pl.when(blk == NBLK - 1)
        def _zovl():
            for o in range(0, OVL, 16):
                gso_v[pl.ds(o, 16)] = zf

        @pl.loop(0, BSTG // 16)
        def off(i):
            src_v[pl.ds(i * 16, 16)] = src_v[pl.ds(i * 16, 16)] + goff

        for pj in range(4):
            pltpu.async_copy(h_hbm.at[src_v.at[pl.ds(pj * CHUNK, CHUNK)]],
                             gbuf.at[pl.ds(pj * CHUNK, CHUNK)], sem_g)

        @pl.loop(0, NCHB)
        def chunk_body(j):
            ga = (j % GRING) * CHUNK
            fa = (j % FRING) * CHUNK
            base = j * CHUNK

            pltpu.make_async_copy(h_hbm.at[pl.ds(0, CHUNK)],
                                  gbuf.at[pl.ds(ga, CHUNK)], sem_g).wait()

            @pl.when(j >= 2)
            def _ws():
                pltpu.make_async_copy(fbuf.at[pl.ds(0, CHUNK)],
                                      acc_sh.at[pl.ds(0, CHUNK)],
                                      sem_s).wait()

            @pl.when(j + 4 < NCHB)
            def _ig():
                nxt = ((j + 4) % GRING) * CHUNK
                pltpu.async_copy(
                    h_hbm.at[src_v.at[pl.ds(base + 4 * CHUNK, CHUNK)]],
                    gbuf.at[pl.ds(nxt, CHUNK)], sem_g)

            gv = gso_v[pl.ds(base, 16)]
            for rr in range(CHUNK):
                gvec = jnp.full((16,), gv[rr], jnp.float32)
                for g in range(FH // 32):
                    v = gbuf[ga + rr, pl.ds(g * 16, 16)]
                    ev = lax.bitcast_convert_type(
                        lax.shift_left(v, 16), jnp.float32)
                    ov = lax.bitcast_convert_type(
                        lax.shift_left(lax.shift_right_logical(v, 16), 16),
                        jnp.float32)
                    fbuf[fa + rr, pl.ds(g * 32, 16)] = ev * gvec
                    fbuf[fa + rr, pl.ds(g * 32 + 16, 16)] = ov * gvec

            pltpu.async_copy(fbuf.at[pl.ds(fa, CHUNK)],
                             acc_sh.at[dst_v.at[pl.ds(base, CHUNK)]],
                             sem_s, add=True)

        pltpu.make_async_copy(fbuf.at[pl.ds(0, CHUNK)],
                              acc_sh.at[pl.ds(0, CHUNK)], sem_s).wait()
        pltpu.make_async_copy(fbuf.at[pl.ds(0, CHUNK)],
                              acc_sh.at[pl.ds(0, CHUNK)], sem_s).wait()

    plsc.subcore_barrier()

    @pl.when(s < NSUB - 1)
    def _cfull():
        pltpu.sync_copy(acc_sh.at[pl.ds(s * ZSLAB, ZSLAB)],
                        out_hbm.at[pl.ds(c * N + s * ZSLAB, ZSLAB)])

    @pl.when(s == NSUB - 1)
    def _clast():
        pltpu.sync_copy(acc_sh.at[pl.ds(s * ZSLAB, ZLAST)],
                        out_hbm.at[pl.ds(c * N + s * ZSLAB, ZLAST)])


@functools.lru_cache(maxsize=1)
def _sc_agg():
    return functools.partial(
        pl.kernel,
        out_type=jax.ShapeDtypeStruct((2 * N, FH), jnp.float32),
        mesh=plsc.VectorSubcoreMesh(core_axis_name="c", subcore_axis_name="s",
                                    num_cores=NCORE, num_subcores=NSUB),
        compiler_params=pltpu.CompilerParams(use_tc_tiling_on_sc=False),
        scratch_types=[
            pltpu.VMEM((BSTG + 8,), jnp.int32),
            pltpu.VMEM((BSTG + 8,), jnp.int32),
            pltpu.VMEM((BSTG + 8,), jnp.float32),
            pltpu.VMEM((GRING * CHUNK, FH // 2), jnp.int32),
            pltpu.VMEM((FRING * CHUNK, FH), jnp.float32),
            pltpu.VMEM_SHARED((N, FH), jnp.float32),
            pltpu.SemaphoreType.DMA,
            pltpu.SemaphoreType.DMA,
        ],
    )(_sc_agg_body)


def _agg_slice(agg_ref, t):
    th, tt = divmod(t, TA // 2)
    return agg_ref[th * N:(th + 1) * N, tt * C1:(tt + 1) * C1]


def _dense1_body(agg_ref, theta_ref, bth_ref, gln2_ref, bln2_ref,
                 w2t_ref, z_ref, ynl_ref):
    inv2 = 1.0 / (N * C1)
    for t in range(TA):
        a = _agg_slice(agg_ref, t)
        y = jnp.dot(a, theta_ref[...], preferred_element_type=jnp.float32)
        y = jnp.maximum(y + bth_ref[...], 0.0)
        mu = jnp.sum(y) * inv2
        var = jnp.sum(y * y) * inv2 - mu * mu
        yn = (y - mu) * lax.rsqrt(var + 1e-12) * gln2_ref[...] + bln2_ref[...]
        zt = jnp.dot(yn, w2t_ref[t * C1:(t + 1) * C1, :],
                     preferred_element_type=jnp.float32)
        if t == 0:
            z_ref[...] = zt
        else:
            z_ref[...] = z_ref[...] + zt
        if t == TA - 1:
            ynl_ref[...] = yn


def _dense2_body(z_ref, ynl_ref, btc2_ref, wal2t_ref, bal2_ref,
                 gln1_ref, bln1_ref, wfc1_ref, bfc1_ref, wfc2_ref, bfc2_ref,
                 out_ref):
    z = z_ref[...] + btc2_ref[...]
    p = z[:, :H0]
    q = z[:, H0:]
    xal = jnp.dot(ynl_ref[...], wal2t_ref[...],
                  preferred_element_type=jnp.float32) + bal2_ref[...]
    zz = (p + xal) * jax.nn.sigmoid(q)
    inv1 = 1.0 / (N * H0)
    mu = jnp.sum(zz) * inv1
    var = jnp.sum(zz * zz) * inv1 - mu * mu
    zn = (zz - mu) * lax.rsqrt(var + 1e-12) * gln1_ref[...] + bln1_ref[...]
    f = jnp.maximum(
        jnp.dot(zn, wfc1_ref[...], preferred_element_type=jnp.float32)
        + bfc1_ref[...], 0.0)
    out_ref[...] = (jnp.dot(f, wfc2_ref[...],
                            preferred_element_type=jnp.float32)
                    + bfc2_ref[...])


def kernel(x, edge_index, gso, W_tc1, b_tc1, W_al1, b_al1, theta, b_theta,
           g_ln2, b_ln2, W_tc2, b_tc2, W_al2, b_al2, g_ln1, b_ln1,
           W_fc1, b_fc1, W_fc2, b_fc2):
    xt = x.reshape(T, N).T
    perm2 = jnp.asarray(np.concatenate([PERM64, 64 + PERM64]))
    perm1 = jnp.asarray(PERM64)
    wct = W_tc1[:, 0, :, 0].T[:, perm2]
    wa1 = W_al1[:, 0][perm1].reshape(1, C1)
    btc1 = b_tc1[perm2]
    bal1 = b_al1[perm1]
    w2t = jnp.transpose(W_tc2[:, :, :, 0], (2, 1, 0)).reshape(FEAT, 2 * H0)
    wal2t = W_al2.T
    wfc2p = jnp.pad(W_fc2, ((0, 0), (0, 8 - OUT)))
    bfc2p = jnp.pad(b_fc2, (0, 8 - OUT)).reshape(1, 8)

    h2 = pl.pallas_call(
        _tgc1_body,
        out_shape=jax.ShapeDtypeStruct((2 * N, FH), jnp.bfloat16),
    )(xt, wct, btc1.reshape(1, 2 * C1), wa1, bal1.reshape(1, C1))
    h2i = lax.bitcast_convert_type(h2.reshape(2 * N, FH // 2, 2), jnp.int32)

    agg2 = _sc_agg()(h2i, edge_index[0], edge_index[1], gso)

    z, ynl = pl.pallas_call(
        _dense1_body,
        out_shape=(jax.ShapeDtypeStruct((N, 2 * H0), jnp.float32),
                   jax.ShapeDtypeStruct((N, C1), jnp.float32)),
    )(agg2, theta, b_theta.reshape(1, C1), g_ln2, b_ln2, w2t)

    outp = pl.pallas_call(
        _dense2_body,
        out_shape=jax.ShapeDtypeStruct((N, 8), jnp.float32),
    )(z, ynl, b_tc2.reshape(1, 2 * H0), wal2t, b_al2.reshape(1, H0),
      g_ln1, b_ln1, W_fc1, b_fc1.reshape(1, H0), wfc2p, bfc2p)

    return outp[:, :OUT].T.reshape(1, OUT, N)


# speedup vs baseline: 1.7698x; 1.7698x over previous
import functools

import jax
import jax.numpy as jnp
from jax import lax
from jax.experimental import pallas as pl
from jax.experimental.pallas import tpu as pltpu
from jax.experimental.pallas import tpu_sc as plsc

N = 10000
E = 160000
T = 8
KT = 3
TA = T - (KT - 1)
C1 = 64
FEAT = TA * C1
FH = FEAT // 2
H0 = 128
OUT = 3

NCORE = 2
NSUB = 16
EPT = E // NSUB
BSTG = 512
NBLK = 20
OVL = NBLK * BSTG - EPT
CHUNK = 16
NCHB = BSTG // CHUNK
NRING = 3
ZSLAB = 632
ZLAST = N - (NSUB - 1) * ZSLAB


def _tgc1_body(xt_ref, wct_ref, bc_ref, wa_ref, ba_ref, out_ref):
    for t in range(TA):
        win = xt_ref[:, t:t + KT]
        y = jnp.dot(win, wct_ref[...], preferred_element_type=jnp.float32)
        y = y + bc_ref[...]
        p = y[:, :C1]
        q = y[:, C1:]
        xal = xt_ref[:, t + KT - 1:t + KT] * wa_ref[...] + ba_ref[...]
        th, tt = divmod(t, TA // 2)
        out_ref[th * N:(th + 1) * N, tt * C1:(tt + 1) * C1] = (
            (p + xal) * jax.nn.sigmoid(q))


def _sc_agg_body(h_hbm, esrc_hbm, edst_hbm, gso_hbm, out_hbm,
                 src_v, dst_v, gso_v, gbuf, acc_sh, sem_g, sem_s):
    c = lax.axis_index("c")
    s = lax.axis_index("s")
    ebase = s * EPT
    goff = c * N

    zf = jnp.zeros((16,), jnp.float32)

    @pl.loop(0, NRING * CHUNK)
    def zrow(r):
        for f in range(FH // 16):
            gbuf[r, pl.ds(f * 16, 16)] = zf

    @pl.when(s < NSUB - 1)
    def _zfull():
        for r0 in range(0, ZSLAB, NRING * CHUNK):
            rl = min(NRING * CHUNK, ZSLAB - r0)
            pltpu.sync_copy(gbuf.at[pl.ds(0, rl)],
                            acc_sh.at[pl.ds(s * ZSLAB + r0, rl)])

    @pl.when(s == NSUB - 1)
    def _zlast():
        for r0 in range(0, ZLAST, NRING * CHUNK):
            rl = min(NRING * CHUNK, ZLAST - r0)
            pltpu.sync_copy(gbuf.at[pl.ds(0, rl)],
                            acc_sh.at[pl.ds(s * ZSLAB + r0, rl)])

    plsc.subcore_barrier()

    @pl.loop(0, NBLK)
    def blk_body(blk):
        bb = ebase + jnp.minimum(blk * BSTG, EPT - BSTG)
        pltpu.sync_copy(esrc_hbm.at[pl.ds(bb, BSTG)], src_v.at[pl.ds(0, BSTG)])
        pltpu.sync_copy(edst_hbm.at[pl.ds(bb, BSTG)], dst_v.at[pl.ds(0, BSTG)])
        pltpu.sync_copy(gso_hbm.at[pl.ds(bb, BSTG)], gso_v.at[pl.ds(0, BSTG)])

        @pl.when(blk == NBLK - 1)
        def _zovl():
            for o in range(0, OVL, 16):
                gso_v[pl.ds(o, 16)] = zf

        @pl.loop(0, NCHB)
        def off(i):
            src_v[pl.ds(i * 16, 16)] = src_v[pl.ds(i * 16, 16)] + goff

        pltpu.async_copy(h_hbm.at[src_v.at[pl.ds(0, CHUNK)]],
                         gbuf.at[pl.ds(0, CHUNK)], sem_g)
        pltpu.async_copy(h_hbm.at[src_v.at[pl.ds(CHUNK, CHUNK)]],
                         gbuf.at[pl.ds(CHUNK, CHUNK)], sem_g)

        @pl.loop(0, NCHB)
        def chunk_body(j):
            a = (j % NRING) * CHUNK
            base = j * CHUNK

            pltpu.make_async_copy(h_hbm.at[pl.ds(0, CHUNK)],
                                  gbuf.at[pl.ds(a, CHUNK)], sem_g).wait()

            @pl.when(j >= 1)
            def _ws():
                pltpu.make_async_copy(gbuf.at[pl.ds(0, CHUNK)],
                                      acc_sh.at[pl.ds(0, CHUNK)],
                                      sem_s).wait()

            @pl.when(j + 2 < NCHB)
            def _ig():
                nxt = ((j + 2) % NRING) * CHUNK
                pltpu.async_copy(
                    h_hbm.at[src_v.at[pl.ds(base + 2 * CHUNK, CHUNK)]],
                    gbuf.at[pl.ds(nxt, CHUNK)], sem_g)

            gv = gso_v[pl.ds(base, 16)]
            for rr in range(16):
                gvec = jnp.full((16,), gv[rr], jnp.float32)
                for f in range(FH // 16):
                    sl = pl.ds(f * 16, 16)
                    gbuf[a + rr, sl] = gbuf[a + rr, sl] * gvec

            pltpu.async_copy(gbuf.at[pl.ds(a, CHUNK)],
                             acc_sh.at[dst_v.at[pl.ds(base, CHUNK)]],
                             sem_s, add=True)

        pltpu.make_async_copy(gbuf.at[pl.ds(0, CHUNK)],
                              acc_sh.at[pl.ds(0, CHUNK)], sem_s).wait()

    plsc.subcore_barrier()

    @pl.when(s < NSUB - 1)
    def _cfull():
        pltpu.sync_copy(acc_sh.at[pl.ds(s * ZSLAB, ZSLAB)],
                        out_hbm.at[pl.ds(c * N + s * ZSLAB, ZSLAB)])

    @pl.when(s == NSUB - 1)
    def _clast():
        pltpu.sync_copy(acc_sh.at[pl.ds(s * ZSLAB, ZLAST)],
                        out_hbm.at[pl.ds(c * N + s * ZSLAB, ZLAST)])


@functools.lru_cache(maxsize=1)
def _sc_agg():
    return functools.partial(
        pl.kernel,
        out_type=jax.ShapeDtypeStruct((2 * N, FH), jnp.float32),
        mesh=plsc.VectorSubcoreMesh(core_axis_name="c", subcore_axis_name="s",
                                    num_cores=NCORE, num_subcores=NSUB),
        compiler_params=pltpu.CompilerParams(use_tc_tiling_on_sc=False),
        scratch_types=[
            pltpu.VMEM((BSTG + 8,), jnp.int32),
            pltpu.VMEM((BSTG + 8,), jnp.int32),
            pltpu.VMEM((BSTG + 8,), jnp.float32),
            pltpu.VMEM((NRING * CHUNK, FH), jnp.float32),
            pltpu.VMEM_SHARED((N, FH), jnp.float32),
            pltpu.SemaphoreType.DMA,
            pltpu.SemaphoreType.DMA,
        ],
    )(_sc_agg_body)


def _agg_slice(agg_ref, t):
    th, tt = divmod(t, TA // 2)
    return agg_ref[th * N:(th + 1) * N, tt * C1:(tt + 1) * C1]


def _dense1_body(agg_ref, theta_ref, bth_ref, gln2_ref, bln2_ref,
                 w2t_ref, z_ref, ynl_ref):
    inv2 = 1.0 / (N * C1)
    for t in range(TA):
        a = _agg_slice(agg_ref, t)
        y = jnp.dot(a, theta_ref[...], preferred_element_type=jnp.float32)
        y = jnp.maximum(y + bth_ref[...], 0.0)
        mu = jnp.sum(y) * inv2
        var = jnp.sum(y * y) * inv2 - mu * mu
        yn = (y - mu) * lax.rsqrt(var + 1e-12) * gln2_ref[...] + bln2_ref[...]
        zt = jnp.dot(yn, w2t_ref[t * C1:(t + 1) * C1, :],
                     preferred_element_type=jnp.float32)
        if t == 0:
            z_ref[...] = zt
        else:
            z_ref[...] = z_ref[...] + zt
        if t == TA - 1:
            ynl_ref[...] = yn


def _dense2_body(z_ref, ynl_ref, btc2_ref, wal2t_ref, bal2_ref,
                 gln1_ref, bln1_ref, wfc1_ref, bfc1_ref, wfc2_ref, bfc2_ref,
                 out_ref):
    z = z_ref[...] + btc2_ref[...]
    p = z[:, :H0]
    q = z[:, H0:]
    xal = jnp.dot(ynl_ref[...], wal2t_ref[...],
                  preferred_element_type=jnp.float32) + bal2_ref[...]
    zz = (p + xal) * jax.nn.sigmoid(q)
    inv1 = 1.0 / (N * H0)
    mu = jnp.sum(zz) * inv1
    var = jnp.sum(zz * zz) * inv1 - mu * mu
    zn = (zz - mu) * lax.rsqrt(var + 1e-12) * gln1_ref[...] + bln1_ref[...]
    f = jnp.maximum(
        jnp.dot(zn, wfc1_ref[...], preferred_element_type=jnp.float32)
        + bfc1_ref[...], 0.0)
    out_ref[...] = (jnp.dot(f, wfc2_ref[...],
                            preferred_element_type=jnp.float32)
                    + bfc2_ref[...])


def kernel(x, edge_index, gso, W_tc1, b_tc1, W_al1, b_al1, theta, b_theta,
           g_ln2, b_ln2, W_tc2, b_tc2, W_al2, b_al2, g_ln1, b_ln1,
           W_fc1, b_fc1, W_fc2, b_fc2):
    xt = x.reshape(T, N).T
    wct = W_tc1[:, 0, :, 0].T
    wa1 = W_al1[:, 0].reshape(1, C1)
    w2t = jnp.transpose(W_tc2[:, :, :, 0], (2, 1, 0)).reshape(FEAT, 2 * H0)
    wal2t = W_al2.T
    wfc2p = jnp.pad(W_fc2, ((0, 0), (0, 8 - OUT)))
    bfc2p = jnp.pad(b_fc2, (0, 8 - OUT)).reshape(1, 8)

    h2 = pl.pallas_call(
        _tgc1_body,
        out_shape=jax.ShapeDtypeStruct((2 * N, FH), jnp.float32),
    )(xt, wct, b_tc1.reshape(1, 2 * C1), wa1, b_al1.reshape(1, C1))

    agg2 = _sc_agg()(h2, edge_index[0], edge_index[1], gso)

    z, ynl = pl.pallas_call(
        _dense1_body,
        out_shape=(jax.ShapeDtypeStruct((N, 2 * H0), jnp.float32),
                   jax.ShapeDtypeStruct((N, C1), jnp.float32)),
    )(agg2, theta, b_theta.reshape(1, C1), g_ln2, b_ln2, w2t)

    outp = pl.pallas_call(
        _dense2_body,
        out_shape=jax.ShapeDtypeStruct((N, 8), jnp.float32),
    )(z, ynl, b_tc2.reshape(1, 2 * H0), wal2t, b_al2.reshape(1, H0),
      g_ln1, b_ln1, W_fc1, b_fc1.reshape(1, H0), wfc2p, bfc2p)

    return outp[:, :OUT].T.reshape(1, OUT, N)
